# baseline (speedup 1.0000x reference)
import functools

import jax
import jax.numpy as jnp
from jax import lax
from jax.experimental import pallas as pl
from jax.experimental.pallas import tpu as pltpu
from jax.experimental.pallas import tpu_sc as plsc

_B = 4096
_S = 200
_D = 64
_DW = _D // 2
_NW = 32
_TOTAL = _B * _S

_GRP = 128
_NGRP = 8
_CHUNK = _GRP * _NGRP
_ROWS_PER_W = _TOTAL // _NW
_NCHUNK = _ROWS_PER_W // _CHUNK

_OUT_ROWS = _TOTAL * _DW // 128
_POS_ROWS = _S * _DW // 128
_BM = 400


def _gather_kernel(idx_hbm, tok_hbm, out_hbm, idx_v, rows_v, sem):
    wid = lax.axis_index("s") * 2 + lax.axis_index("c")

    def chunk_body(c, carry):
        base = pl.multiple_of(wid * _ROWS_PER_W + c * _CHUNK, _CHUNK)

        irow = pl.multiple_of(base // _GRP, _NGRP)
        pltpu.sync_copy(idx_hbm.at[pl.ds(irow, _NGRP)], idx_v)

        copies = []
        for j in range(_NGRP):
            copies.append(
                pltpu.async_copy(
                    tok_hbm.at[idx_v.at[j]],
                    rows_v.at[pl.ds(j * _GRP, _GRP)],
                    sem,
                )
            )
        for cp in copies:
            cp.wait()

        pltpu.sync_copy(rows_v, out_hbm.at[pl.ds(base, _CHUNK)])
        return carry

    lax.fori_loop(0, _NCHUNK, chunk_body, 0)


def _add_kernel(g_ref, pos_ref, o_ref):
    g = pltpu.bitcast(g_ref[...], jnp.bfloat16)
    p = pltpu.bitcast(pos_ref[...], jnp.bfloat16)
    p_tiled = jnp.concatenate([p] * (_BM // _POS_ROWS), axis=0)
    o_ref[...] = pltpu.bitcast(g + p_tiled, jnp.int32)


@jax.jit
def kernel(x, token_table, pos_table):
    tok_i32 = lax.bitcast_convert_type(
        token_table.reshape(token_table.shape[0], _DW, 2), jnp.int32
    )
    pos_i32 = lax.bitcast_convert_type(
        pos_table[:_S].reshape(_S, _DW, 2), jnp.int32
    ).reshape(_POS_ROWS, 128)
    idx = x.reshape(_TOTAL // _GRP, _GRP).astype(jnp.int32)

    mesh = plsc.VectorSubcoreMesh(core_axis_name="c", subcore_axis_name="s")
    g = pl.kernel(
        _gather_kernel,
        mesh=mesh,
        compiler_params=pltpu.CompilerParams(use_tc_tiling_on_sc=False),
        out_type=jax.ShapeDtypeStruct((_TOTAL, _DW), jnp.int32),
        scratch_types=[
            pltpu.VMEM((_NGRP, _GRP), jnp.int32),
            pltpu.VMEM((_CHUNK, _DW), jnp.int32),
            pltpu.SemaphoreType.DMA,
        ],
    )(idx, tok_i32)

    g2 = g.reshape(_OUT_ROWS, 128)
    out = pl.pallas_call(
        _add_kernel,
        grid=(_OUT_ROWS // _BM,),
        in_specs=[
            pl.BlockSpec((_BM, 128), lambda i: (i, 0)),
            pl.BlockSpec((_POS_ROWS, 128), lambda i: (0, 0)),
        ],
        out_specs=pl.BlockSpec((_BM, 128), lambda i: (i, 0)),
        out_shape=jax.ShapeDtypeStruct((_OUT_ROWS, 128), jnp.int32),
    )(g2, pos_i32)

    e = lax.bitcast_convert_type(out, jnp.bfloat16)
    return e.reshape(_B, _S, _D)


# speedup vs baseline: 2.3537x; 2.3537x over previous
import functools

import jax
import jax.numpy as jnp
from jax import lax
from jax.experimental import pallas as pl
from jax.experimental.pallas import tpu as pltpu
from jax.experimental.pallas import tpu_sc as plsc

_B = 4096
_S = 200
_D = 64
_DW = _D // 2
_NW = 32
_TOTAL = _B * _S

_GRP = 128
_NGRP = 8
_CHUNK = _GRP * _NGRP
_ROWS_PER_W = _TOTAL // _NW
_NCHUNK = _ROWS_PER_W // _CHUNK

_OUT_ROWS = _TOTAL * _DW // 128
_POS_ROWS = _S * _DW // 128
_BM = 400


def _gather_kernel(idx_hbm, tok_hbm, out_hbm, idx_v, rows_v, sem):
    wid = lax.axis_index("s") * 2 + lax.axis_index("c")

    def chunk_body(c, carry):
        base = pl.multiple_of(wid * _ROWS_PER_W + c * _CHUNK, _CHUNK)

        irow = pl.multiple_of(base // _GRP, _NGRP)
        pltpu.sync_copy(idx_hbm.at[pl.ds(irow, _NGRP)], idx_v)

        copies = []
        for j in range(_NGRP):
            copies.append(
                pltpu.async_copy(
                    tok_hbm.at[idx_v.at[j]],
                    rows_v.at[pl.ds(j * _GRP, _GRP)],
                    sem,
                )
            )
        for cp in copies:
            cp.wait()

        pltpu.sync_copy(rows_v, out_hbm.at[pl.ds(base, _CHUNK)])
        return carry

    lax.fori_loop(0, _NCHUNK, chunk_body, 0)


def _add_kernel(g_ref, pos_ref, o_ref):
    p = pltpu.bitcast(g_ref[...], jnp.bfloat16)
    pe = p.reshape(_BM, 2, 4, _DW)
    blk = jnp.transpose(pe, (0, 2, 3, 1)).reshape(4 * _BM, _D)
    p_tiled = jnp.concatenate([pos_ref[...]] * (4 * _BM // _S), axis=0)
    o_ref[...] = blk + p_tiled


@jax.jit
def kernel(x, token_table, pos_table):
    tok_i32 = lax.bitcast_convert_type(
        token_table.reshape(token_table.shape[0], _DW, 2), jnp.int32
    )
    pos_b = pos_table[:_S]
    idx = x.reshape(_TOTAL // _GRP, _GRP).astype(jnp.int32)

    mesh = plsc.VectorSubcoreMesh(core_axis_name="c", subcore_axis_name="s")
    g = pl.kernel(
        _gather_kernel,
        mesh=mesh,
        compiler_params=pltpu.CompilerParams(use_tc_tiling_on_sc=False),
        out_type=jax.ShapeDtypeStruct((_TOTAL, _DW), jnp.int32),
        scratch_types=[
            pltpu.VMEM((_NGRP, _GRP), jnp.int32),
            pltpu.VMEM((_CHUNK, _DW), jnp.int32),
            pltpu.SemaphoreType.DMA,
        ],
    )(idx, tok_i32)

    g2 = g.reshape(_OUT_ROWS, 128)
    out = pl.pallas_call(
        _add_kernel,
        grid=(_OUT_ROWS // _BM,),
        in_specs=[
            pl.BlockSpec((_BM, 128), lambda i: (i, 0)),
            pl.BlockSpec((_S, _D), lambda i: (0, 0)),
        ],
        out_specs=pl.BlockSpec((4 * _BM, _D), lambda i: (i, 0)),
        out_shape=jax.ShapeDtypeStruct((_TOTAL, _D), jnp.bfloat16),
    )(g2, pos_b)

    return out.reshape(_B, _S, _D)


# speedup vs baseline: 14.9035x; 6.3320x over previous
import functools

import jax
import jax.numpy as jnp
from jax import lax
from jax.experimental import pallas as pl
from jax.experimental.pallas import tpu as pltpu
from jax.experimental.pallas import tpu_sc as plsc

_B = 4096
_S = 200
_D = 64
_DW = _D // 2
_NW = 32
_TOTAL = _B * _S

_GRP = 128
_NGRP = 8
_CHUNK = _GRP * _NGRP
_ROWS_PER_W = _TOTAL // _NW
_NCHUNK = _ROWS_PER_W // _CHUNK

_OUT_ROWS = _TOTAL * _DW // 128
_POS_ROWS = _S * _DW // 128
_BM = 400


def _gather_kernel(idx_hbm, tok_hbm, out_hbm, idx_v, rows_v, sem):
    wid = lax.axis_index("s") * 2 + lax.axis_index("c")

    def chunk_body(c, carry):
        base = pl.multiple_of(wid * _ROWS_PER_W + c * _CHUNK, _CHUNK)

        irow = pl.multiple_of(base // _GRP, _NGRP)
        pltpu.sync_copy(idx_hbm.at[pl.ds(irow, _NGRP)], idx_v)

        copies = []
        for j in range(_NGRP):
            copies.append(
                pltpu.async_copy(
                    tok_hbm.at[idx_v.at[j]],
                    rows_v.at[pl.ds(j * _GRP, _GRP)],
                    sem,
                )
            )
        for cp in copies:
            cp.wait()

        pltpu.sync_copy(rows_v, out_hbm.at[pl.ds(base, _CHUNK)])
        return carry

    lax.fori_loop(0, _NCHUNK, chunk_body, 0)


def _add_kernel(g_ref, pos_ref, o_ref):
    p = pltpu.bitcast(g_ref[...], jnp.bfloat16)
    pe = p.reshape(_BM, 2, 4, _DW)
    blk = jnp.transpose(pe, (0, 2, 1, 3)).reshape(4 * _BM, _D)
    p_tiled = jnp.concatenate([pos_ref[...]] * (4 * _BM // _S), axis=0)
    o_ref[...] = blk + p_tiled


@jax.jit
def kernel(x, token_table, pos_table):
    tok_de = jnp.swapaxes(
        token_table.reshape(token_table.shape[0], 2, _DW), 1, 2
    )
    tok_i32 = lax.bitcast_convert_type(tok_de, jnp.int32)
    pos_b = pos_table[:_S]
    idx = x.reshape(_TOTAL // _GRP, _GRP).astype(jnp.int32)

    mesh = plsc.VectorSubcoreMesh(core_axis_name="c", subcore_axis_name="s")
    g = pl.kernel(
        _gather_kernel,
        mesh=mesh,
        compiler_params=pltpu.CompilerParams(use_tc_tiling_on_sc=False),
        out_type=jax.ShapeDtypeStruct((_TOTAL, _DW), jnp.int32),
        scratch_types=[
            pltpu.VMEM((_NGRP, _GRP), jnp.int32),
            pltpu.VMEM((_CHUNK, _DW), jnp.int32),
            pltpu.SemaphoreType.DMA,
        ],
    )(idx, tok_i32)

    g2 = g.reshape(_OUT_ROWS, 128)
    out = pl.pallas_call(
        _add_kernel,
        grid=(_OUT_ROWS // _BM,),
        in_specs=[
            pl.BlockSpec((_BM, 128), lambda i: (i, 0)),
            pl.BlockSpec((_S, _D), lambda i: (0, 0)),
        ],
        out_specs=pl.BlockSpec((4 * _BM, _D), lambda i: (i, 0)),
        out_shape=jax.ShapeDtypeStruct((_TOTAL, _D), jnp.bfloat16),
    )(g2, pos_b)

    return out.reshape(_B, _S, _D)
